# baseline (speedup 1.0000x reference)
import jax
import jax.numpy as jnp
from jax import lax
from jax.experimental import pallas as pl

_ALPHA = 0.5
_ROW_BLOCK = 8192


def _linear_kernel(x_ref, w_in_ref, b_in_ref, w_out_ref, b_out_ref, o_ref):
    w = _ALPHA * w_in_ref[...] + (1.0 - _ALPHA) * w_out_ref[...]
    b = _ALPHA * b_in_ref[...] + (1.0 - _ALPHA) * b_out_ref[...]
    acc = lax.dot_general(
        x_ref[...], w,
        dimension_numbers=(((1,), (1,)), ((), ())),
        preferred_element_type=jnp.float32,
    )
    o_ref[...] = acc + b[None, :]


def kernel(x, At, W_in, b_in, W_out, b_out):
    del At
    Bd, Nd, L = x.shape
    out_ch = W_in.shape[0]
    rows = Bd * Nd
    xf = x.reshape(rows, L)

    grid = (rows // _ROW_BLOCK,)
    out = pl.pallas_call(
        _linear_kernel,
        grid=grid,
        in_specs=[
            pl.BlockSpec((_ROW_BLOCK, L), lambda i: (i, 0)),
            pl.BlockSpec((out_ch, L), lambda i: (0, 0)),
            pl.BlockSpec((out_ch,), lambda i: (0,)),
            pl.BlockSpec((out_ch, L), lambda i: (0, 0)),
            pl.BlockSpec((out_ch,), lambda i: (0,)),
        ],
        out_specs=pl.BlockSpec((_ROW_BLOCK, out_ch), lambda i: (i, 0)),
        out_shape=jax.ShapeDtypeStruct((rows, out_ch), jnp.float32),
    )(xf, W_in, b_in, W_out, b_out)
    return out.reshape(Bd, Nd, out_ch)


# speedup vs baseline: 1.0893x; 1.0893x over previous
import jax
import jax.numpy as jnp
from jax import lax
from jax.experimental import pallas as pl

_ALPHA = 0.5
_ROW_BLOCK = 4096


def _linear_kernel(x_ref, w_in_ref, b_in_ref, w_out_ref, b_out_ref, o_ref):
    w = _ALPHA * w_in_ref[...] + (1.0 - _ALPHA) * w_out_ref[...]
    b = _ALPHA * b_in_ref[...] + (1.0 - _ALPHA) * b_out_ref[...]
    acc = lax.dot_general(
        x_ref[...], w,
        dimension_numbers=(((1,), (1,)), ((), ())),
        preferred_element_type=jnp.float32,
    )
    o_ref[...] = acc + b[None, :]


def kernel(x, At, W_in, b_in, W_out, b_out):
    del At
    Bd, Nd, L = x.shape
    out_ch = W_in.shape[0]
    rows = Bd * Nd
    xf = x.reshape(rows, L)

    grid = (rows // _ROW_BLOCK,)
    out = pl.pallas_call(
        _linear_kernel,
        grid=grid,
        in_specs=[
            pl.BlockSpec((_ROW_BLOCK, L), lambda i: (i, 0)),
            pl.BlockSpec((out_ch, L), lambda i: (0, 0)),
            pl.BlockSpec((out_ch,), lambda i: (0,)),
            pl.BlockSpec((out_ch, L), lambda i: (0, 0)),
            pl.BlockSpec((out_ch,), lambda i: (0,)),
        ],
        out_specs=pl.BlockSpec((_ROW_BLOCK, out_ch), lambda i: (i, 0)),
        out_shape=jax.ShapeDtypeStruct((rows, out_ch), jnp.float32),
    )(xf, W_in, b_in, W_out, b_out)
    return out.reshape(Bd, Nd, out_ch)
